# Initial kernel scaffold; baseline (speedup 1.0000x reference)
import jax
import jax.numpy as jnp
from jax.experimental import pallas as pl


def kernel(query, key, value, omega, alpha, beta):
    raise NotImplementedError("write your pallas kernel here")


# speedup vs baseline: 4.4121x; 4.4121x over previous
import functools
import math

import jax
import jax.numpy as jnp
from jax import lax
from jax.experimental import pallas as pl
from jax.experimental.pallas import tpu as pltpu
from jax.experimental.pallas import tpu_sc as plsc

B, S, H, E = 2, 2048, 16, 64
BH = B * H
M = 64
NH = 2
BK = 128
NB = S // BK
TEMP = 1.0 / math.sqrt(E)
NRM = math.sqrt(TEMP)
LOGM = math.log(M)
PAW = 80
QW = 128
KW = 128
UW = 128
NW = 32
CH = 128


def _mm(a, b, ca, cb):
    return lax.dot_general(a, b, (((ca,), (cb,)), ((), ())),
                           precision=lax.Precision.HIGHEST,
                           preferred_element_type=jnp.float32)


def _stats_body(q_ref, k_ref, v_ref, om_ref, pa_ref):
    q = q_ref[0]
    k = k_ref[0]
    v = v_ref[0]
    om = om_ref[...]
    qsq = jnp.sum(q * q, axis=-1, keepdims=True)
    ksq = jnp.sum(k * k, axis=-1, keepdims=True)
    qdd = _mm(q, om, 1, 1) * NRM - qsq * (TEMP / 2.0)
    kdd = _mm(k, om, 1, 1) * NRM - ksq * (TEMP / 2.0)
    qstab = jnp.max(qdd, axis=-1, keepdims=True)
    kstab = jnp.max(kdd)
    qp = jnp.exp(qdd - qstab)
    kp = jnp.exp(kdd - kstab)
    kls = kstab - LOGM
    prime_ls = qstab + kls
    ksum = jnp.sum(kp, axis=0, keepdims=True)
    kv = _mm(kp, v, 0, 0)
    qkv = _mm(qp, kv, 1, 0)
    qk1 = _mm(qp, ksum, 1, 1)
    ones = jnp.ones((S, 1), jnp.float32)
    pa_ref[0] = jnp.concatenate(
        [qkv, qk1, prime_ls, ones * kls, ones * kstab,
         jnp.zeros((S, PAW - E - 4), jnp.float32)], axis=1)


def _run_stats(q, k, v, omega):
    return pl.pallas_call(
        _stats_body,
        grid=(BH,),
        in_specs=[
            pl.BlockSpec((1, S, E), lambda b: (b, 0, 0)),
            pl.BlockSpec((1, S, E), lambda b: (b, 0, 0)),
            pl.BlockSpec((1, S, E), lambda b: (b, 0, 0)),
            pl.BlockSpec((M, E), lambda b: (0, 0)),
        ],
        out_specs=pl.BlockSpec((1, S, PAW), lambda b: (b, 0, 0)),
        out_shape=jax.ShapeDtypeStruct((BH, S, PAW), jnp.float32),
    )(q, k, v, omega)


def _sc_gather(table, idx, d):
    r_rows = idx.shape[0]
    per_w = r_rows // NW
    n_ch = per_w // CH
    mesh = plsc.VectorSubcoreMesh(core_axis_name="c", subcore_axis_name="s")

    @functools.partial(
        pl.kernel, mesh=mesh,
        out_type=jax.ShapeDtypeStruct((r_rows, d), jnp.float32),
        scratch_types=[
            pltpu.VMEM((CH,), jnp.int32),
            pltpu.VMEM((CH, d), jnp.float32),
            pltpu.SemaphoreType.DMA,
        ],
    )
    def gk(table_hbm, idx_hbm, out_hbm, idx_v, rows_v, sem):
        wid = lax.axis_index("s") * 2 + lax.axis_index("c")
        base = wid * per_w

        def body(c, carry):
            off = base + c * CH
            pltpu.sync_copy(idx_hbm.at[pl.ds(off, CH)], idx_v)
            pltpu.async_copy(table_hbm.at[idx_v], rows_v, sem).wait()
            pltpu.sync_copy(rows_v, out_hbm.at[pl.ds(off, CH)])
            return carry

        lax.fori_loop(0, n_ch, body, 0)

    return gk(table, idx)


def _bucket_body(qt_ref, kt_ref, cq_ref, ck_ref, om_ref, ut_ref):
    qt = qt_ref[...]
    kt = kt_ref[...]
    om = om_ref[...]
    sq = qt[:, :E]
    skls = qt[:, E:E + 1]
    skstab = qt[:, E + 1:E + 2]
    sk = kt[:, :E]
    sv = kt[:, E:2 * E]
    inner = _mm(sq, sk, 1, 1) * TEMP
    qsq = jnp.sum(sq * sq, axis=-1, keepdims=True)
    ksq = jnp.sum(sk * sk, axis=-1, keepdims=True)
    qdd = _mm(sq, om, 1, 1) * NRM - qsq * (TEMP / 2.0)
    qstab = jnp.max(qdd, axis=-1, keepdims=True)
    sqp = jnp.exp(qdd - qstab)
    sls = qstab + skls
    kdd = _mm(sk, om, 1, 1) * NRM - ksq * (TEMP / 2.0)
    skp = jnp.exp(kdd - skstab)
    dots_prime = _mm(sqp, skp, 1, 1)
    dup = 1.0 + (jnp.transpose(cq_ref[0]) == ck_ref[0]).astype(jnp.float32)
    inner = inner - jnp.log(dup)
    dots_prime = dots_prime / dup
    lse = jnp.maximum(jnp.max(inner, axis=-1, keepdims=True), sls)
    dots = jnp.exp(inner - lse) - dots_prime * jnp.exp(sls - lse)
    dsum = jnp.sum(dots, axis=-1, keepdims=True)
    so = _mm(dots, sv, 1, 0)
    ut_ref[...] = jnp.concatenate(
        [so, lse, dsum, jnp.zeros((BK, UW - E - 2), jnp.float32)], axis=1)


def _run_buckets(SQ, SK, CQ, CK, omega):
    n = NH * BH * NB
    return pl.pallas_call(
        _bucket_body,
        grid=(n,),
        in_specs=[
            pl.BlockSpec((BK, QW), lambda p: (p, 0)),
            pl.BlockSpec((BK, KW), lambda p: (p, 0)),
            pl.BlockSpec((1, 1, BK), lambda p: (p, 0, 0)),
            pl.BlockSpec((1, 1, BK), lambda p: (p, 0, 0)),
            pl.BlockSpec((M, E), lambda p: (0, 0)),
        ],
        out_specs=pl.BlockSpec((BK, UW), lambda p: (p, 0)),
        out_shape=jax.ShapeDtypeStruct((n * BK, UW), jnp.float32),
    )(SQ, SK, CQ, CK, omega)


def _combine_body(u0_ref, u1_ref, pa_ref, o_ref):
    u0 = u0_ref[...]
    u1 = u1_ref[...]
    pa = pa_ref[0]
    o0, l0, s0 = u0[:, :E], u0[:, E:E + 1], u0[:, E + 1:E + 2]
    o1, l1, s1 = u1[:, :E], u1[:, E:E + 1], u1[:, E + 1:E + 2]
    mx = jnp.maximum(l0, l1)
    nls = mx + jnp.log(jnp.exp(l0 - mx) + jnp.exp(l1 - mx))
    p0 = jnp.exp(l0 - nls)
    p1 = jnp.exp(l1 - nls)
    osm = o0 * p0 + o1 * p1
    dst = s0 * p0 + s1 * p1
    psc = jnp.exp(pa[:, E + 1:E + 2] - nls)
    out = osm + pa[:, :E] * psc
    nz = dst + pa[:, E:E + 1] * psc
    o_ref[...] = out / jnp.maximum(nz, 1e-6)


def _run_combine(U, PA):
    return pl.pallas_call(
        _combine_body,
        grid=(BH,),
        in_specs=[
            pl.BlockSpec((S, UW), lambda b: (b, 0)),
            pl.BlockSpec((S, UW), lambda b: (BH + b, 0)),
            pl.BlockSpec((1, S, PAW), lambda b: (b, 0, 0)),
        ],
        out_specs=pl.BlockSpec((S, E), lambda b: (b, 0)),
        out_shape=jax.ShapeDtypeStruct((BH * S, E), jnp.float32),
    )(U, U, PA)


def kernel(query, key, value, omega, alpha, beta):
    q = jnp.transpose(query, (0, 2, 1, 3)).reshape(BH, S, E)
    k = jnp.transpose(key, (0, 2, 1, 3)).reshape(BH, S, E)
    v = jnp.transpose(value, (0, 2, 1, 3)).reshape(BH, S, E)

    PA = _run_stats(q, k, v, omega)

    q_norms = jnp.linalg.norm(q, axis=-1, keepdims=True)
    k_norms = jnp.linalg.norm(k, axis=-1, keepdims=True)
    MQ = jnp.max(q_norms, axis=-2, keepdims=True)
    MK = jnp.max(k_norms, axis=-2, keepdims=True)
    ext_q = jnp.sqrt(jnp.maximum(MQ**2 - q_norms**2, 0.0))
    ext_k = jnp.sqrt(jnp.maximum(MK**2 - k_norms**2, 0.0))
    q_t = jnp.concatenate([q, ext_q, jnp.zeros_like(ext_q)], axis=-1)
    k_t = jnp.concatenate([k, jnp.zeros_like(ext_k), ext_k], axis=-1)
    qh = jnp.transpose(jnp.einsum('bsd,dh->bsh', q_t, alpha) + beta, (2, 0, 1))
    kh = jnp.transpose(jnp.einsum('bsd,dh->bsh', k_t, alpha) + beta, (2, 0, 1))
    q_pos = jnp.argsort(qh, axis=-1).astype(jnp.int32)
    k_pos = jnp.argsort(kh, axis=-1).astype(jnp.int32)
    iota = jnp.broadcast_to(jnp.arange(S, dtype=jnp.int32), (NH, BH, S))
    hix = jnp.arange(NH, dtype=jnp.int32)[:, None, None]
    bix = jnp.arange(BH, dtype=jnp.int32)[None, :, None]
    q_rev = jnp.zeros((NH, BH, S), jnp.int32).at[hix, bix, q_pos].set(iota)
    k_rev = jnp.zeros((NH, BH, S), jnp.int32).at[hix, bix, k_pos].set(iota)
    cq = (jnp.take_along_axis(q_rev[::-1], q_pos, axis=-1) // BK)
    ck = (jnp.take_along_axis(k_rev[::-1], k_pos, axis=-1) // BK)
    CQ = cq.astype(jnp.float32).reshape(NH * BH * NB, 1, BK)
    CK = ck.astype(jnp.float32).reshape(NH * BH * NB, 1, BK)

    kls_col = jnp.broadcast_to(PA[:, :, E + 2:E + 3], (BH, S, 1))
    kst_col = jnp.broadcast_to(PA[:, :, E + 3:E + 4], (BH, S, 1))
    QT = jnp.concatenate(
        [q, kls_col, kst_col, jnp.zeros((BH, S, QW - E - 2), jnp.float32)],
        axis=-1).reshape(BH * S, QW)
    KT = jnp.concatenate([k, v], axis=-1).reshape(BH * S, KW)

    boff = (jnp.arange(BH, dtype=jnp.int32) * S)[None, :, None]
    gq = (boff + q_pos).reshape(-1)
    gk_ = (boff + k_pos).reshape(-1)
    SQ = _sc_gather(QT, gq, QW)
    SK = _sc_gather(KT, gk_, KW)

    UT = _run_buckets(SQ, SK, CQ, CK, omega)

    ubase = (jnp.arange(NH, dtype=jnp.int32) * (BH * S))[:, None, None] + boff
    gu = (ubase + q_rev).reshape(-1)
    U = _sc_gather(UT, gu, UW)

    res = _run_combine(U, PA)
    return jnp.transpose(res.reshape(B, H, S, E), (0, 2, 1, 3))


# speedup vs baseline: 5.0039x; 1.1341x over previous
import functools
import math

import jax
import jax.numpy as jnp
from jax import lax
from jax.experimental import pallas as pl
from jax.experimental.pallas import tpu as pltpu
from jax.experimental.pallas import tpu_sc as plsc

B, S, H, E = 2, 2048, 16, 64
BH = B * H
M = 64
NH = 2
BK = 128
NB = S // BK
TEMP = 1.0 / math.sqrt(E)
NRM = math.sqrt(TEMP)
LOGM = math.log(M)
PAW = 80
QW = 128
KW = 128
UW = 128
NW = 32
CH = 128


def _mm(a, b, ca, cb):
    return lax.dot_general(a, b, (((ca,), (cb,)), ((), ())),
                           precision=lax.Precision.HIGHEST,
                           preferred_element_type=jnp.float32)


def _stats_body(q_ref, k_ref, v_ref, om_ref, pa_ref):
    q = q_ref[0]
    k = k_ref[0]
    v = v_ref[0]
    om = om_ref[...]
    qsq = jnp.sum(q * q, axis=-1, keepdims=True)
    ksq = jnp.sum(k * k, axis=-1, keepdims=True)
    qdd = _mm(q, om, 1, 1) * NRM - qsq * (TEMP / 2.0)
    kdd = _mm(k, om, 1, 1) * NRM - ksq * (TEMP / 2.0)
    qstab = jnp.max(qdd, axis=-1, keepdims=True)
    kstab = jnp.max(kdd)
    qp = jnp.exp(qdd - qstab)
    kp = jnp.exp(kdd - kstab)
    kls = kstab - LOGM
    prime_ls = qstab + kls
    ksum = jnp.sum(kp, axis=0, keepdims=True)
    kv = _mm(kp, v, 0, 0)
    qkv = _mm(qp, kv, 1, 0)
    qk1 = _mm(qp, ksum, 1, 1)
    ones = jnp.ones((S, 1), jnp.float32)
    pa_ref[0] = jnp.concatenate(
        [qkv, qk1, prime_ls, ones * kls, ones * kstab,
         jnp.zeros((S, PAW - E - 4), jnp.float32)], axis=1)


def _run_stats(q, k, v, omega):
    return pl.pallas_call(
        _stats_body,
        grid=(BH,),
        in_specs=[
            pl.BlockSpec((1, S, E), lambda b: (b, 0, 0)),
            pl.BlockSpec((1, S, E), lambda b: (b, 0, 0)),
            pl.BlockSpec((1, S, E), lambda b: (b, 0, 0)),
            pl.BlockSpec((M, E), lambda b: (0, 0)),
        ],
        out_specs=pl.BlockSpec((1, S, PAW), lambda b: (b, 0, 0)),
        out_shape=jax.ShapeDtypeStruct((BH, S, PAW), jnp.float32),
    )(q, k, v, omega)


def _sc_gather(table, idx, d):
    r_rows = idx.shape[0]
    per_w = r_rows // NW
    n_ch = per_w // CH
    mesh = plsc.VectorSubcoreMesh(core_axis_name="c", subcore_axis_name="s")

    @functools.partial(
        pl.kernel, mesh=mesh,
        out_type=jax.ShapeDtypeStruct((r_rows, d), jnp.float32),
        scratch_types=[
            pltpu.VMEM((CH,), jnp.int32),
            pltpu.VMEM((CH, d), jnp.float32),
            pltpu.SemaphoreType.DMA,
        ],
    )
    def gk(table_hbm, idx_hbm, out_hbm, idx_v, rows_v, sem):
        wid = lax.axis_index("s") * 2 + lax.axis_index("c")
        base = wid * per_w

        def body(c, carry):
            off = base + c * CH
            pltpu.sync_copy(idx_hbm.at[pl.ds(off, CH)], idx_v)
            pltpu.async_copy(table_hbm.at[idx_v], rows_v, sem).wait()
            pltpu.sync_copy(rows_v, out_hbm.at[pl.ds(off, CH)])
            return carry

        lax.fori_loop(0, n_ch, body, 0)

    return gk(table, idx)


GB = 8


def _bucket_body(qt_ref, kt_ref, cq_ref, ck_ref, om_ref, ut_ref):
    om = om_ref[...]
    for g in range(GB):
        _one_bucket(qt_ref[g * BK:(g + 1) * BK], kt_ref[g * BK:(g + 1) * BK],
                    cq_ref[g], ck_ref[g], om,
                    ut_ref.at[g * BK:(g + 1) * BK])


def _one_bucket(qt, kt, cq, ck, om, ut_ref):
    sq = qt[:, :E]
    skls = qt[:, E:E + 1]
    skstab = qt[:, E + 1:E + 2]
    sk = kt[:, :E]
    sv = kt[:, E:2 * E]
    inner = _mm(sq, sk, 1, 1) * TEMP
    qsq = jnp.sum(sq * sq, axis=-1, keepdims=True)
    ksq = jnp.sum(sk * sk, axis=-1, keepdims=True)
    qdd = _mm(sq, om, 1, 1) * NRM - qsq * (TEMP / 2.0)
    qstab = jnp.max(qdd, axis=-1, keepdims=True)
    sqp = jnp.exp(qdd - qstab)
    sls = qstab + skls
    kdd = _mm(sk, om, 1, 1) * NRM - ksq * (TEMP / 2.0)
    skp = jnp.exp(kdd - skstab)
    dots_prime = _mm(sqp, skp, 1, 1)
    dup = 1.0 + (jnp.transpose(cq) == ck).astype(jnp.float32)
    inner = inner - jnp.log(dup)
    dots_prime = dots_prime / dup
    lse = jnp.maximum(jnp.max(inner, axis=-1, keepdims=True), sls)
    dots = jnp.exp(inner - lse) - dots_prime * jnp.exp(sls - lse)
    dsum = jnp.sum(dots, axis=-1, keepdims=True)
    so = _mm(dots, sv, 1, 0)
    ut_ref[...] = jnp.concatenate(
        [so, lse, dsum, jnp.zeros((BK, UW - E - 2), jnp.float32)], axis=1)


_ONE_BUCKET_DONE = True


def _run_buckets(SQ, SK, CQ, CK, omega):
    n = NH * BH * NB
    return pl.pallas_call(
        _bucket_body,
        grid=(n // GB,),
        in_specs=[
            pl.BlockSpec((GB * BK, QW), lambda p: (p, 0)),
            pl.BlockSpec((GB * BK, KW), lambda p: (p, 0)),
            pl.BlockSpec((GB, 1, BK), lambda p: (p, 0, 0)),
            pl.BlockSpec((GB, 1, BK), lambda p: (p, 0, 0)),
            pl.BlockSpec((M, E), lambda p: (0, 0)),
        ],
        out_specs=pl.BlockSpec((GB * BK, UW), lambda p: (p, 0)),
        out_shape=jax.ShapeDtypeStruct((n * BK, UW), jnp.float32),
    )(SQ, SK, CQ, CK, omega)


def _combine_body(u0_ref, u1_ref, pa_ref, o_ref):
    u0 = u0_ref[...]
    u1 = u1_ref[...]
    pa = pa_ref[0]
    o0, l0, s0 = u0[:, :E], u0[:, E:E + 1], u0[:, E + 1:E + 2]
    o1, l1, s1 = u1[:, :E], u1[:, E:E + 1], u1[:, E + 1:E + 2]
    mx = jnp.maximum(l0, l1)
    nls = mx + jnp.log(jnp.exp(l0 - mx) + jnp.exp(l1 - mx))
    p0 = jnp.exp(l0 - nls)
    p1 = jnp.exp(l1 - nls)
    osm = o0 * p0 + o1 * p1
    dst = s0 * p0 + s1 * p1
    psc = jnp.exp(pa[:, E + 1:E + 2] - nls)
    out = osm + pa[:, :E] * psc
    nz = dst + pa[:, E:E + 1] * psc
    o_ref[...] = out / jnp.maximum(nz, 1e-6)


def _run_combine(U, PA):
    return pl.pallas_call(
        _combine_body,
        grid=(BH,),
        in_specs=[
            pl.BlockSpec((S, UW), lambda b: (b, 0)),
            pl.BlockSpec((S, UW), lambda b: (BH + b, 0)),
            pl.BlockSpec((1, S, PAW), lambda b: (b, 0, 0)),
        ],
        out_specs=pl.BlockSpec((S, E), lambda b: (b, 0)),
        out_shape=jax.ShapeDtypeStruct((BH * S, E), jnp.float32),
    )(U, U, PA)


def kernel(query, key, value, omega, alpha, beta):
    q = jnp.transpose(query, (0, 2, 1, 3)).reshape(BH, S, E)
    k = jnp.transpose(key, (0, 2, 1, 3)).reshape(BH, S, E)
    v = jnp.transpose(value, (0, 2, 1, 3)).reshape(BH, S, E)

    PA = _run_stats(q, k, v, omega)

    q_norms = jnp.linalg.norm(q, axis=-1, keepdims=True)
    k_norms = jnp.linalg.norm(k, axis=-1, keepdims=True)
    MQ = jnp.max(q_norms, axis=-2, keepdims=True)
    MK = jnp.max(k_norms, axis=-2, keepdims=True)
    ext_q = jnp.sqrt(jnp.maximum(MQ**2 - q_norms**2, 0.0))
    ext_k = jnp.sqrt(jnp.maximum(MK**2 - k_norms**2, 0.0))
    q_t = jnp.concatenate([q, ext_q, jnp.zeros_like(ext_q)], axis=-1)
    k_t = jnp.concatenate([k, jnp.zeros_like(ext_k), ext_k], axis=-1)
    qh = jnp.transpose(jnp.einsum('bsd,dh->bsh', q_t, alpha) + beta, (2, 0, 1))
    kh = jnp.transpose(jnp.einsum('bsd,dh->bsh', k_t, alpha) + beta, (2, 0, 1))
    q_pos = jnp.argsort(qh, axis=-1).astype(jnp.int32)
    k_pos = jnp.argsort(kh, axis=-1).astype(jnp.int32)
    iota = jnp.broadcast_to(jnp.arange(S, dtype=jnp.int32), (NH, BH, S))
    hix = jnp.arange(NH, dtype=jnp.int32)[:, None, None]
    bix = jnp.arange(BH, dtype=jnp.int32)[None, :, None]
    q_rev = jnp.zeros((NH, BH, S), jnp.int32).at[hix, bix, q_pos].set(iota)
    k_rev = jnp.zeros((NH, BH, S), jnp.int32).at[hix, bix, k_pos].set(iota)
    cq = (jnp.take_along_axis(q_rev[::-1], q_pos, axis=-1) // BK)
    ck = (jnp.take_along_axis(k_rev[::-1], k_pos, axis=-1) // BK)
    CQ = cq.astype(jnp.float32).reshape(NH * BH * NB, 1, BK)
    CK = ck.astype(jnp.float32).reshape(NH * BH * NB, 1, BK)

    kls_col = jnp.broadcast_to(PA[:, :, E + 2:E + 3], (BH, S, 1))
    kst_col = jnp.broadcast_to(PA[:, :, E + 3:E + 4], (BH, S, 1))
    QT = jnp.concatenate(
        [q, kls_col, kst_col, jnp.zeros((BH, S, QW - E - 2), jnp.float32)],
        axis=-1).reshape(BH * S, QW)
    KT = jnp.concatenate([k, v], axis=-1).reshape(BH * S, KW)

    boff = (jnp.arange(BH, dtype=jnp.int32) * S)[None, :, None]
    gq = (boff + q_pos).reshape(-1)
    gk_ = (boff + k_pos).reshape(-1)
    SQ = _sc_gather(QT, gq, QW)
    SK = _sc_gather(KT, gk_, KW)

    UT = _run_buckets(SQ, SK, CQ, CK, omega)

    ubase = (jnp.arange(NH, dtype=jnp.int32) * (BH * S))[:, None, None] + boff
    gu = (ubase + q_rev).reshape(-1)
    U = _sc_gather(UT, gu, UW)

    res = _run_combine(U, PA)
    return jnp.transpose(res.reshape(B, H, S, E), (0, 2, 1, 3))


# speedup vs baseline: 5.0671x; 1.0126x over previous
import functools
import math

import jax
import jax.numpy as jnp
from jax import lax
from jax.experimental import pallas as pl
from jax.experimental.pallas import tpu as pltpu
from jax.experimental.pallas import tpu_sc as plsc

B, S, H, E = 2, 2048, 16, 64
BH = B * H
M = 64
NH = 2
BK = 128
NB = S // BK
TEMP = 1.0 / math.sqrt(E)
NRM = math.sqrt(TEMP)
LOGM = math.log(M)
PAW = 80
QW = 128
KW = 128
UW = 128
NW = 32
CH = 128


def _mm(a, b, ca, cb):
    return lax.dot_general(a, b, (((ca,), (cb,)), ((), ())),
                           precision=lax.Precision.HIGHEST,
                           preferred_element_type=jnp.float32)


def _stats_body(q_ref, k_ref, v_ref, om_ref, pa_ref):
    q = q_ref[0]
    k = k_ref[0]
    v = v_ref[0]
    om = om_ref[...]
    qsq = jnp.sum(q * q, axis=-1, keepdims=True)
    ksq = jnp.sum(k * k, axis=-1, keepdims=True)
    qdd = _mm(q, om, 1, 1) * NRM - qsq * (TEMP / 2.0)
    kdd = _mm(k, om, 1, 1) * NRM - ksq * (TEMP / 2.0)
    qstab = jnp.max(qdd, axis=-1, keepdims=True)
    kstab = jnp.max(kdd)
    qp = jnp.exp(qdd - qstab)
    kp = jnp.exp(kdd - kstab)
    kls = kstab - LOGM
    prime_ls = qstab + kls
    ksum = jnp.sum(kp, axis=0, keepdims=True)
    kv = _mm(kp, v, 0, 0)
    qkv = _mm(qp, kv, 1, 0)
    qk1 = _mm(qp, ksum, 1, 1)
    ones = jnp.ones((S, 1), jnp.float32)
    pa_ref[0] = jnp.concatenate(
        [qkv, qk1, prime_ls, ones * kls, ones * kstab,
         jnp.zeros((S, PAW - E - 4), jnp.float32)], axis=1)


def _run_stats(q, k, v, omega):
    return pl.pallas_call(
        _stats_body,
        grid=(BH,),
        in_specs=[
            pl.BlockSpec((1, S, E), lambda b: (b, 0, 0)),
            pl.BlockSpec((1, S, E), lambda b: (b, 0, 0)),
            pl.BlockSpec((1, S, E), lambda b: (b, 0, 0)),
            pl.BlockSpec((M, E), lambda b: (0, 0)),
        ],
        out_specs=pl.BlockSpec((1, S, PAW), lambda b: (b, 0, 0)),
        out_shape=jax.ShapeDtypeStruct((BH, S, PAW), jnp.float32),
    )(q, k, v, omega)


def _sc_gather(table, idx, d):
    r_rows = idx.shape[0]
    per_w = r_rows // NW
    n_ch = per_w // CH
    mesh = plsc.VectorSubcoreMesh(core_axis_name="c", subcore_axis_name="s")

    @functools.partial(
        pl.kernel, mesh=mesh,
        out_type=jax.ShapeDtypeStruct((r_rows, d), jnp.float32),
        scratch_types=[
            pltpu.VMEM((CH,), jnp.int32),
            pltpu.VMEM((CH, d), jnp.float32),
            pltpu.SemaphoreType.DMA,
        ],
    )
    def gk(table_hbm, idx_hbm, out_hbm, idx_v, rows_v, sem):
        wid = lax.axis_index("s") * 2 + lax.axis_index("c")
        base = wid * per_w

        def body(c, carry):
            off = base + c * CH
            pltpu.sync_copy(idx_hbm.at[pl.ds(off, CH)], idx_v)
            pltpu.async_copy(table_hbm.at[idx_v], rows_v, sem).wait()
            pltpu.sync_copy(rows_v, out_hbm.at[pl.ds(off, CH)])
            return carry

        lax.fori_loop(0, n_ch, body, 0)

    return gk(table, idx)


GB = 16


def _bucket_body(qt_ref, kt_ref, cq_ref, ck_ref, om_ref, ut_ref):
    om = om_ref[...]
    for g in range(GB):
        _one_bucket(qt_ref[g * BK:(g + 1) * BK], kt_ref[g * BK:(g + 1) * BK],
                    cq_ref[g], ck_ref[g], om,
                    ut_ref.at[g * BK:(g + 1) * BK])


def _one_bucket(qt, kt, cq, ck, om, ut_ref):
    sq = qt[:, :E]
    skls = qt[:, E:E + 1]
    skstab = qt[:, E + 1:E + 2]
    sk = kt[:, :E]
    sv = kt[:, E:2 * E]
    inner = _mm(sq, sk, 1, 1) * TEMP
    qsq = jnp.sum(sq * sq, axis=-1, keepdims=True)
    ksq = jnp.sum(sk * sk, axis=-1, keepdims=True)
    qdd = _mm(sq, om, 1, 1) * NRM - qsq * (TEMP / 2.0)
    qstab = jnp.max(qdd, axis=-1, keepdims=True)
    sqp = jnp.exp(qdd - qstab)
    sls = qstab + skls
    kdd = _mm(sk, om, 1, 1) * NRM - ksq * (TEMP / 2.0)
    skp = jnp.exp(kdd - skstab)
    dots_prime = _mm(sqp, skp, 1, 1)
    dup = 1.0 + (jnp.transpose(cq) == ck).astype(jnp.float32)
    inner = inner - jnp.log(dup)
    dots_prime = dots_prime / dup
    lse = jnp.maximum(jnp.max(inner, axis=-1, keepdims=True), sls)
    dots = jnp.exp(inner - lse) - dots_prime * jnp.exp(sls - lse)
    dsum = jnp.sum(dots, axis=-1, keepdims=True)
    so = _mm(dots, sv, 1, 0)
    ut_ref[...] = jnp.concatenate(
        [so, lse, dsum, jnp.zeros((BK, UW - E - 2), jnp.float32)], axis=1)


_ONE_BUCKET_DONE = True


def _run_buckets(SQ, SK, CQ, CK, omega):
    n = NH * BH * NB
    return pl.pallas_call(
        _bucket_body,
        grid=(n // GB,),
        in_specs=[
            pl.BlockSpec((GB * BK, QW), lambda p: (p, 0)),
            pl.BlockSpec((GB * BK, KW), lambda p: (p, 0)),
            pl.BlockSpec((GB, 1, BK), lambda p: (p, 0, 0)),
            pl.BlockSpec((GB, 1, BK), lambda p: (p, 0, 0)),
            pl.BlockSpec((M, E), lambda p: (0, 0)),
        ],
        out_specs=pl.BlockSpec((GB * BK, UW), lambda p: (p, 0)),
        out_shape=jax.ShapeDtypeStruct((n * BK, UW), jnp.float32),
    )(SQ, SK, CQ, CK, omega)


def _combine_body(u0_ref, u1_ref, pa_ref, o_ref):
    u0 = u0_ref[...]
    u1 = u1_ref[...]
    pa = pa_ref[0]
    o0, l0, s0 = u0[:, :E], u0[:, E:E + 1], u0[:, E + 1:E + 2]
    o1, l1, s1 = u1[:, :E], u1[:, E:E + 1], u1[:, E + 1:E + 2]
    mx = jnp.maximum(l0, l1)
    nls = mx + jnp.log(jnp.exp(l0 - mx) + jnp.exp(l1 - mx))
    p0 = jnp.exp(l0 - nls)
    p1 = jnp.exp(l1 - nls)
    osm = o0 * p0 + o1 * p1
    dst = s0 * p0 + s1 * p1
    psc = jnp.exp(pa[:, E + 1:E + 2] - nls)
    out = osm + pa[:, :E] * psc
    nz = dst + pa[:, E:E + 1] * psc
    o_ref[...] = out / jnp.maximum(nz, 1e-6)


def _run_combine(U, PA):
    return pl.pallas_call(
        _combine_body,
        grid=(BH,),
        in_specs=[
            pl.BlockSpec((S, UW), lambda b: (b, 0)),
            pl.BlockSpec((S, UW), lambda b: (BH + b, 0)),
            pl.BlockSpec((1, S, PAW), lambda b: (b, 0, 0)),
        ],
        out_specs=pl.BlockSpec((S, E), lambda b: (b, 0)),
        out_shape=jax.ShapeDtypeStruct((BH * S, E), jnp.float32),
    )(U, U, PA)


def kernel(query, key, value, omega, alpha, beta):
    q = jnp.transpose(query, (0, 2, 1, 3)).reshape(BH, S, E)
    k = jnp.transpose(key, (0, 2, 1, 3)).reshape(BH, S, E)
    v = jnp.transpose(value, (0, 2, 1, 3)).reshape(BH, S, E)

    PA = _run_stats(q, k, v, omega)

    q_norms = jnp.linalg.norm(q, axis=-1, keepdims=True)
    k_norms = jnp.linalg.norm(k, axis=-1, keepdims=True)
    MQ = jnp.max(q_norms, axis=-2, keepdims=True)
    MK = jnp.max(k_norms, axis=-2, keepdims=True)
    ext_q = jnp.sqrt(jnp.maximum(MQ**2 - q_norms**2, 0.0))
    ext_k = jnp.sqrt(jnp.maximum(MK**2 - k_norms**2, 0.0))
    q_t = jnp.concatenate([q, ext_q, jnp.zeros_like(ext_q)], axis=-1)
    k_t = jnp.concatenate([k, jnp.zeros_like(ext_k), ext_k], axis=-1)
    qh = jnp.transpose(jnp.einsum('bsd,dh->bsh', q_t, alpha) + beta, (2, 0, 1))
    kh = jnp.transpose(jnp.einsum('bsd,dh->bsh', k_t, alpha) + beta, (2, 0, 1))
    q_pos = jnp.argsort(qh, axis=-1).astype(jnp.int32)
    k_pos = jnp.argsort(kh, axis=-1).astype(jnp.int32)
    iota = jnp.broadcast_to(jnp.arange(S, dtype=jnp.int32), (NH, BH, S))
    hix = jnp.arange(NH, dtype=jnp.int32)[:, None, None]
    bix = jnp.arange(BH, dtype=jnp.int32)[None, :, None]
    q_rev = jnp.zeros((NH, BH, S), jnp.int32).at[hix, bix, q_pos].set(iota)
    k_rev = jnp.zeros((NH, BH, S), jnp.int32).at[hix, bix, k_pos].set(iota)
    cq = (jnp.take_along_axis(q_rev[::-1], q_pos, axis=-1) // BK)
    ck = (jnp.take_along_axis(k_rev[::-1], k_pos, axis=-1) // BK)
    CQ = cq.astype(jnp.float32).reshape(NH * BH * NB, 1, BK)
    CK = ck.astype(jnp.float32).reshape(NH * BH * NB, 1, BK)

    kls_col = jnp.broadcast_to(PA[:, :, E + 2:E + 3], (BH, S, 1))
    kst_col = jnp.broadcast_to(PA[:, :, E + 3:E + 4], (BH, S, 1))
    QT = jnp.concatenate(
        [q, kls_col, kst_col, jnp.zeros((BH, S, QW - E - 2), jnp.float32)],
        axis=-1).reshape(BH * S, QW)
    KT = jnp.concatenate([k, v], axis=-1).reshape(BH * S, KW)

    boff = (jnp.arange(BH, dtype=jnp.int32) * S)[None, :, None]
    gq = (boff + q_pos).reshape(-1)
    gk_ = (boff + k_pos).reshape(-1)
    SQ = _sc_gather(QT, gq, QW)
    SK = _sc_gather(KT, gk_, KW)

    UT = _run_buckets(SQ, SK, CQ, CK, omega)

    ubase = (jnp.arange(NH, dtype=jnp.int32) * (BH * S))[:, None, None] + boff
    gu = (ubase + q_rev).reshape(-1)
    U = _sc_gather(UT, gu, UW)

    res = _run_combine(U, PA)
    return jnp.transpose(res.reshape(B, H, S, E), (0, 2, 1, 3))
